# baseline (speedup 1.0000x reference)
import jax
import jax.numpy as jnp
from jax import lax
from jax.experimental import pallas as pl
from jax.experimental.pallas import tpu as pltpu
from jax.experimental.pallas import tpu_sc as plsc

N = 20000
NPAD = 160 * 128
ROWS = 160
LANES = 128
PRE_NMS_TOPK = 1000
MAX_DET = 300
IOU_THRESH = 0.5
SCORE_THRESH = 0.05

NSUB = 16
SLICE = NPAD // NSUB
VPT = SLICE // 16
CAP = 2048
OUTN = 2 * CAP
CROWS = CAP // LANES


def _thresh_kernel(s_ref, t_ref):
    s = s_ref[...]
    sb = lax.bitcast_convert_type(s, jnp.int32)

    def bs_step(_, carry):
        lo, hi = carry
        mid = lo + ((hi - lo) >> 1)
        cnt = jnp.sum((sb >= mid).astype(jnp.int32))
        ge = cnt >= PRE_NMS_TOPK
        lo = jnp.where(ge, mid, lo)
        hi = jnp.where(ge, hi, mid)
        return lo, hi

    lo, _ = lax.fori_loop(0, 31, bs_step, (jnp.int32(0), jnp.int32(0x7F800000)))
    t_ref[...] = jnp.full((8, LANES), lo, jnp.int32)


def _sc_compact(s_hbm, x1_hbm, y1_hbm, x2_hbm, y2_hbm, t_hbm,
                s_out, x1_out, y1_out, x2_out, y2_out, o_out,
                sv, x1v, y1v, x2v, y2v,
                tv, rankv, idxv, cntbuf, ov, zv, ctr,
                s_sh, x1_sh, y1_sh, x2_sh, y2_sh, o_sh, sem):
    wid = lax.axis_index("s")
    base = wid * SLICE
    pltpu.sync_copy(s_hbm.at[pl.ds(base, SLICE)], sv)
    pltpu.sync_copy(x1_hbm.at[pl.ds(base, SLICE)], x1v)
    pltpu.sync_copy(y1_hbm.at[pl.ds(base, SLICE)], y1v)
    pltpu.sync_copy(x2_hbm.at[pl.ds(base, SLICE)], x2v)
    pltpu.sync_copy(y2_hbm.at[pl.ds(base, SLICE)], y2v)
    pltpu.sync_copy(t_hbm, tv)
    t = tv[...]
    thr = jnp.full((16,), SCORE_THRESH, jnp.float32)
    lane16 = lax.iota(jnp.int32, 16)
    ones = jnp.ones((16,), jnp.int32)
    zeros = jnp.zeros((16,), jnp.int32)

    @pl.when(wid == 0)
    def _init():
        ctr[0] = 0

    @pl.when(wid < 5)
    def _zero_fill():
        def zstep(i, _):
            zv[pl.ds(i * 16, 16)] = jnp.zeros((16,), jnp.float32)
            return 0

        lax.fori_loop(0, CAP // 16, zstep, 0)

    for k, sh in enumerate((s_sh, x1_sh, y1_sh, x2_sh, y2_sh)):
        @pl.when(wid == k)
        def _zcp(sh=sh):
            pltpu.sync_copy(zv, sh.at[pl.ds(0, CAP)])

    BIG = jnp.int32(1 << 20)
    bigv = jnp.full((16,), BIG, jnp.int32)

    basev = jnp.broadcast_to(base, (16,))

    def comp_step(i, cnt):
        sl = pl.ds(i * 16, 16)
        s16 = sv[sl]
        m = (s16 >= t) & (s16 > thr)
        mi = jnp.where(m, ones, zeros)
        rankv[sl] = jnp.where(m, cnt, bigv)
        ov[sl] = basev + jnp.broadcast_to(i * 16, (16,)) + lane16
        return cnt + mi

    cnt = lax.fori_loop(0, VPT, comp_step, zeros)

    n_local = jnp.int32(0)
    pvec = zeros
    for j in range(16):
        ej = cnt[j]
        n_local = n_local + ej
        pvec = pvec + jnp.where(lane16 > j, jnp.broadcast_to(ej, (16,)), zeros)

    plsc.subcore_barrier()
    g = plsc.fetch_and_add(ctr.at[0], n_local, subcore_id=0)
    gp = jnp.broadcast_to(g, (16,)) + pvec

    for i in range(VPT):
        sl = pl.ds(i * 16, 16)
        r16 = rankv[sl]
        dump = jnp.full((16,), CAP + i * 16, jnp.int32) + lane16
        dst = jnp.where(r16 < bigv, gp + r16, dump)
        idxv[i // 8, pl.ds((i % 8) * 16, 16)] = dst

    copies = []
    for src, dst in ((sv, s_sh), (x1v, x1_sh), (y1v, y1_sh),
                     (x2v, x2_sh), (y2v, y2_sh), (ov, o_sh)):
        for c in range(SLICE // 128):
            copies.append(
                pltpu.async_copy(src.at[pl.ds(c * 128, 128)],
                                 dst.at[idxv.at[c]], sem))
    for cp in copies:
        cp.wait()

    plsc.subcore_barrier()

    for k, (sh, out) in enumerate(((s_sh, s_out), (x1_sh, x1_out),
                                   (y1_sh, y1_out), (x2_sh, x2_out),
                                   (y2_sh, y2_out), (o_sh, o_out))):
        @pl.when(wid == k)
        def _flush(sh=sh, out=out):
            pltpu.sync_copy(sh.at[pl.ds(0, CAP)], out.at[pl.ds(0, CAP)])


def _nms_kernel(x1_ref, y1_ref, x2_ref, y2_ref, s_ref, o_ref, out_ref):
    x1 = x1_ref[...]
    y1 = y1_ref[...]
    x2 = x2_ref[...]
    y2 = y2_ref[...]
    s = s_ref[...]
    orig = o_ref[...]
    sm0 = jnp.where(s > SCORE_THRESH, s, -1.0)

    areas = (x2 - x1) * (y2 - y1)
    lane = lax.broadcasted_iota(jnp.int32, (1, LANES), 1)

    def nms_step(k, sm):
        m = jnp.max(sm)
        any_valid = m > 0.0
        eq = sm == m
        idx = jnp.min(jnp.where(eq & any_valid, orig, jnp.int32(NPAD)))
        sel = orig == idx
        bx1 = jnp.sum(jnp.where(sel, x1, 0.0))
        by1 = jnp.sum(jnp.where(sel, y1, 0.0))
        bx2 = jnp.sum(jnp.where(sel, x2, 0.0))
        by2 = jnp.sum(jnp.where(sel, y2, 0.0))
        bs_ = jnp.where(any_valid, m, 0.0)

        xx1 = jnp.maximum(bx1, x1)
        yy1 = jnp.maximum(by1, y1)
        xx2 = jnp.minimum(bx2, x2)
        yy2 = jnp.minimum(by2, y2)
        inter = jnp.maximum(xx2 - xx1, 0.0) * jnp.maximum(yy2 - yy1, 0.0)
        barea = (bx2 - bx1) * (by2 - by1)
        iou = inter / (barea + areas - inter + 1e-9)
        sm = jnp.where(iou < IOU_THRESH, sm, -1.0)

        row = jnp.where(lane == 0, bx1, 0.0)
        row = jnp.where(lane == 1, by1, row)
        row = jnp.where(lane == 2, bx2, row)
        row = jnp.where(lane == 3, by2, row)
        row = jnp.where(lane == 4, bs_, row)
        out_ref[pl.ds(k, 1), :] = row
        return sm

    lax.fori_loop(0, MAX_DET, nms_step, sm0)


def kernel(boxes, scores):
    pad = NPAD - N
    s_flat = jnp.concatenate([scores, jnp.full((pad,), -1.0, jnp.float32)])
    b = jnp.concatenate([boxes, jnp.zeros((pad, 4), jnp.float32)], axis=0)

    tq = pl.pallas_call(
        _thresh_kernel,
        out_shape=jax.ShapeDtypeStruct((8, LANES), jnp.int32),
    )(s_flat.reshape(ROWS, LANES))
    t16 = jnp.broadcast_to(lax.bitcast_convert_type(tq[0, 0], jnp.float32), (16,))

    mesh = plsc.VectorSubcoreMesh(
        core_axis_name="c", subcore_axis_name="s", num_cores=1)
    f32 = jnp.float32
    sc_outs = pl.kernel(
        _sc_compact,
        out_type=[jax.ShapeDtypeStruct((OUTN,), f32)] * 5
        + [jax.ShapeDtypeStruct((OUTN,), jnp.int32)],
        mesh=mesh,
        scratch_types=[pltpu.VMEM((SLICE,), f32)] * 5 + [
            pltpu.VMEM((16,), jnp.float32),
            pltpu.VMEM((SLICE,), jnp.int32),
            pltpu.VMEM((SLICE // 128, 128), jnp.int32),
            pltpu.VMEM((16,), jnp.int32),
            pltpu.VMEM((SLICE,), jnp.int32),
            pltpu.VMEM((CAP,), f32),
            pltpu.SMEM((1,), jnp.int32),
        ] + [pltpu.VMEM_SHARED((OUTN,), f32)] * 5 + [
            pltpu.VMEM_SHARED((OUTN,), jnp.int32),
            pltpu.SemaphoreType.DMA,
        ],
    )(s_flat, b[:, 0], b[:, 1], b[:, 2], b[:, 3], t16)
    s_c, x1_c, y1_c, x2_c, y2_c, o_c = [a[:CAP].reshape(CROWS, LANES)
                                        for a in sc_outs]

    out = pl.pallas_call(
        _nms_kernel,
        out_shape=jax.ShapeDtypeStruct((304, LANES), jnp.float32),
    )(x1_c, y1_c, x2_c, y2_c, s_c, o_c)
    return out[:MAX_DET, :5]


# speedup vs baseline: 1.3305x; 1.3305x over previous
import jax
import jax.numpy as jnp
from jax import lax
from jax.experimental import pallas as pl
from jax.experimental.pallas import tpu as pltpu
from jax.experimental.pallas import tpu_sc as plsc

N = 20000
NPAD = 160 * 128
ROWS = 160
LANES = 128
PRE_NMS_TOPK = 1000
MAX_DET = 300
IOU_THRESH = 0.5
SCORE_THRESH = 0.05

NSUB = 16
SLICE = NPAD // NSUB
VPT = SLICE // 16
CAP = 2048
OUTN = 2 * CAP
CROWS = CAP // LANES


def _thresh_kernel(s_ref, t_ref):
    s = s_ref[...]
    sb = lax.bitcast_convert_type(s, jnp.int32)

    def bs_step(_, carry):
        lo, hi = carry
        mid = lo + ((hi - lo) >> 1)
        cnt = jnp.sum((sb >= mid).astype(jnp.int32))
        ge = cnt >= PRE_NMS_TOPK
        lo = jnp.where(ge, mid, lo)
        hi = jnp.where(ge, hi, mid)
        return lo, hi

    lo, _ = lax.fori_loop(0, 31, bs_step, (jnp.int32(0), jnp.int32(0x7F800000)))
    t_ref[...] = jnp.full((8, LANES), lo, jnp.int32)


def _sc_compact(s_hbm, x1_hbm, y1_hbm, x2_hbm, y2_hbm, t_hbm,
                s_out, x1_out, y1_out, x2_out, y2_out, o_out,
                sv, x1v, y1v, x2v, y2v,
                tv, rankv, idxv, cntbuf, ov, zv, ctr,
                s_sh, x1_sh, y1_sh, x2_sh, y2_sh, o_sh, sem):
    wid = lax.axis_index("s")
    base = wid * SLICE
    pltpu.sync_copy(s_hbm.at[pl.ds(base, SLICE)], sv)
    pltpu.sync_copy(x1_hbm.at[pl.ds(base, SLICE)], x1v)
    pltpu.sync_copy(y1_hbm.at[pl.ds(base, SLICE)], y1v)
    pltpu.sync_copy(x2_hbm.at[pl.ds(base, SLICE)], x2v)
    pltpu.sync_copy(y2_hbm.at[pl.ds(base, SLICE)], y2v)
    pltpu.sync_copy(t_hbm, tv)
    t = tv[...]
    thr = jnp.full((16,), SCORE_THRESH, jnp.float32)
    lane16 = lax.iota(jnp.int32, 16)
    ones = jnp.ones((16,), jnp.int32)
    zeros = jnp.zeros((16,), jnp.int32)

    @pl.when(wid == 0)
    def _init():
        ctr[0] = 0

    @pl.when(wid < 5)
    def _zero_fill():
        def zstep(i, _):
            zv[pl.ds(i * 16, 16)] = jnp.zeros((16,), jnp.float32)
            return 0

        lax.fori_loop(0, CAP // 16, zstep, 0)

    for k, sh in enumerate((s_sh, x1_sh, y1_sh, x2_sh, y2_sh)):
        @pl.when(wid == k)
        def _zcp(sh=sh):
            pltpu.sync_copy(zv, sh.at[pl.ds(0, CAP)])

    BIG = jnp.int32(1 << 20)
    bigv = jnp.full((16,), BIG, jnp.int32)

    basev = jnp.broadcast_to(base, (16,))

    def comp_step(i, cnt):
        sl = pl.ds(i * 16, 16)
        s16 = sv[sl]
        m = (s16 >= t) & (s16 > thr)
        mi = jnp.where(m, ones, zeros)
        rankv[sl] = jnp.where(m, cnt, bigv)
        ov[sl] = basev + jnp.broadcast_to(i * 16, (16,)) + lane16
        return cnt + mi

    cnt = lax.fori_loop(0, VPT, comp_step, zeros)

    n_local = jnp.int32(0)
    pvec = zeros
    for j in range(16):
        ej = cnt[j]
        n_local = n_local + ej
        pvec = pvec + jnp.where(lane16 > j, jnp.broadcast_to(ej, (16,)), zeros)

    plsc.subcore_barrier()
    g = plsc.fetch_and_add(ctr.at[0], n_local, subcore_id=0)
    gp = jnp.broadcast_to(g, (16,)) + pvec

    for i in range(VPT):
        sl = pl.ds(i * 16, 16)
        r16 = rankv[sl]
        dump = jnp.full((16,), CAP + i * 16, jnp.int32) + lane16
        dst = jnp.where(r16 < bigv, gp + r16, dump)
        idxv[i // 8, pl.ds((i % 8) * 16, 16)] = dst

    copies = []
    for src, dst in ((sv, s_sh), (x1v, x1_sh), (y1v, y1_sh),
                     (x2v, x2_sh), (y2v, y2_sh), (ov, o_sh)):
        for c in range(SLICE // 128):
            copies.append(
                pltpu.async_copy(src.at[pl.ds(c * 128, 128)],
                                 dst.at[idxv.at[c]], sem))
    for cp in copies:
        cp.wait()

    plsc.subcore_barrier()

    for k, (sh, out) in enumerate(((s_sh, s_out), (x1_sh, x1_out),
                                   (y1_sh, y1_out), (x2_sh, x2_out),
                                   (y2_sh, y2_out), (o_sh, o_out))):
        @pl.when(wid == k)
        def _flush(sh=sh, out=out):
            pltpu.sync_copy(sh.at[pl.ds(0, CAP)], out.at[pl.ds(0, CAP)])


def _nms_kernel(x1_ref, y1_ref, x2_ref, y2_ref, s_ref, o_ref, data_ref,
                x1s_ref, y1s_ref, x2s_ref, y2s_ref, out_ref):
    x1 = x1_ref[...]
    y1 = y1_ref[...]
    x2 = x2_ref[...]
    y2 = y2_ref[...]
    s = s_ref[...]
    orig = o_ref[...]
    sm0 = jnp.where(s > SCORE_THRESH, s, -1.0)

    areas = (x2 - x1) * (y2 - y1)
    ii = lax.broadcasted_iota(jnp.int32, (CROWS, LANES), 0)
    jj = lax.broadcasted_iota(jnp.int32, (CROWS, LANES), 1)
    flat = ii * LANES + jj
    key = (orig << 11) | flat
    BIG = jnp.int32(0x7FFFFFFF)

    def nms_step(k, sm):
        m_b = jnp.max(jnp.max(sm, axis=1, keepdims=True), axis=0,
                      keepdims=True)
        eqv = (sm == m_b) & (m_b > 0.0)
        keym = jnp.where(eqv, key, BIG)
        kmin = jnp.min(jnp.min(keym, axis=1, keepdims=True), axis=0,
                       keepdims=True)
        p = kmin[0, 0] & jnp.int32(CAP - 1)
        bx1 = x1s_ref[p]
        by1 = y1s_ref[p]
        bx2 = x2s_ref[p]
        by2 = y2s_ref[p]

        xx1 = jnp.maximum(bx1, x1)
        yy1 = jnp.maximum(by1, y1)
        xx2 = jnp.minimum(bx2, x2)
        yy2 = jnp.minimum(by2, y2)
        inter = jnp.maximum(xx2 - xx1, 0.0) * jnp.maximum(yy2 - yy1, 0.0)
        barea = (bx2 - bx1) * (by2 - by1)
        denom = barea + areas - inter + 1e-9
        sm = jnp.where(2.0 * inter < denom, sm, -1.0)

        row8 = data_ref[pl.ds(p, 1), :]
        out_ref[pl.ds(k, 1), :] = row8
        return sm

    lax.fori_loop(0, MAX_DET, nms_step, sm0)


def kernel(boxes, scores):
    pad = NPAD - N
    s_flat = jnp.concatenate([scores, jnp.full((pad,), -1.0, jnp.float32)])
    b = jnp.concatenate([boxes, jnp.zeros((pad, 4), jnp.float32)], axis=0)

    tq = pl.pallas_call(
        _thresh_kernel,
        out_shape=jax.ShapeDtypeStruct((8, LANES), jnp.int32),
    )(s_flat.reshape(ROWS, LANES))
    t16 = jnp.broadcast_to(lax.bitcast_convert_type(tq[0, 0], jnp.float32), (16,))

    mesh = plsc.VectorSubcoreMesh(
        core_axis_name="c", subcore_axis_name="s", num_cores=1)
    f32 = jnp.float32
    sc_outs = pl.kernel(
        _sc_compact,
        out_type=[jax.ShapeDtypeStruct((OUTN,), f32)] * 5
        + [jax.ShapeDtypeStruct((OUTN,), jnp.int32)],
        mesh=mesh,
        scratch_types=[pltpu.VMEM((SLICE,), f32)] * 5 + [
            pltpu.VMEM((16,), jnp.float32),
            pltpu.VMEM((SLICE,), jnp.int32),
            pltpu.VMEM((SLICE // 128, 128), jnp.int32),
            pltpu.VMEM((16,), jnp.int32),
            pltpu.VMEM((SLICE,), jnp.int32),
            pltpu.VMEM((CAP,), f32),
            pltpu.SMEM((1,), jnp.int32),
        ] + [pltpu.VMEM_SHARED((OUTN,), f32)] * 5 + [
            pltpu.VMEM_SHARED((OUTN,), jnp.int32),
            pltpu.SemaphoreType.DMA,
        ],
    )(s_flat, b[:, 0], b[:, 1], b[:, 2], b[:, 3], t16)
    s_c, x1_c, y1_c, x2_c, y2_c, o_c = [a[:CAP].reshape(CROWS, LANES)
                                        for a in sc_outs]
    cols = [sc_outs[1][:CAP], sc_outs[2][:CAP], sc_outs[3][:CAP],
            sc_outs[4][:CAP], sc_outs[0][:CAP]]
    data8 = jnp.stack(cols + [jnp.zeros((CAP,), f32)] * 3, axis=1)

    vmem = pl.BlockSpec(memory_space=pltpu.MemorySpace.VMEM)
    smem = pl.BlockSpec(memory_space=pltpu.MemorySpace.SMEM)
    out = pl.pallas_call(
        _nms_kernel,
        out_shape=jax.ShapeDtypeStruct((304, 8), jnp.float32),
        in_specs=[vmem] * 7 + [smem] * 4,
    )(x1_c, y1_c, x2_c, y2_c, s_c, o_c, data8,
      sc_outs[1][:CAP], sc_outs[2][:CAP], sc_outs[3][:CAP], sc_outs[4][:CAP])
    return out[:MAX_DET, :5]


# speedup vs baseline: 1.5291x; 1.1493x over previous
import jax
import jax.numpy as jnp
from jax import lax
from jax.experimental import pallas as pl
from jax.experimental.pallas import tpu as pltpu
from jax.experimental.pallas import tpu_sc as plsc

N = 20000
NPAD = 160 * 128
ROWS = 160
LANES = 128
PRE_NMS_TOPK = 1000
MAX_DET = 300
IOU_THRESH = 0.5
SCORE_THRESH = 0.05

NSUB = 16
SLICE = NPAD // NSUB
VPT = SLICE // 16
CAP = 2048
OUTN = 2 * CAP
CROWS = CAP // LANES


def _thresh_kernel(s_ref, t_ref):
    s = s_ref[...]
    sb = lax.bitcast_convert_type(s, jnp.int32)

    def bs_step(_, carry):
        lo, hi = carry
        mid = lo + ((hi - lo) >> 1)
        cnt = jnp.sum((sb >= mid).astype(jnp.int32))
        ge = cnt >= PRE_NMS_TOPK
        lo = jnp.where(ge, mid, lo)
        hi = jnp.where(ge, hi, mid)
        return lo, hi

    lo, _ = lax.fori_loop(0, 31, bs_step, (jnp.int32(0), jnp.int32(0x7F800000)))
    t_ref[...] = jnp.full((8, LANES), lo, jnp.int32)


def _sc_compact(s_hbm, x1_hbm, y1_hbm, x2_hbm, y2_hbm, t_hbm,
                s_out, x1_out, y1_out, x2_out, y2_out,
                sv, x1v, y1v, x2v, y2v,
                tv, rankv, idxv, cntbuf, cloc, cnt_all, zv,
                s_sh, x1_sh, y1_sh, x2_sh, y2_sh, cnt_sh, sem):
    wid = lax.axis_index("s")
    base = wid * SLICE
    pltpu.sync_copy(s_hbm.at[pl.ds(base, SLICE)], sv)
    pltpu.sync_copy(x1_hbm.at[pl.ds(base, SLICE)], x1v)
    pltpu.sync_copy(y1_hbm.at[pl.ds(base, SLICE)], y1v)
    pltpu.sync_copy(x2_hbm.at[pl.ds(base, SLICE)], x2v)
    pltpu.sync_copy(y2_hbm.at[pl.ds(base, SLICE)], y2v)
    pltpu.sync_copy(t_hbm, tv)
    t = tv[...]
    thr = jnp.full((16,), SCORE_THRESH, jnp.float32)
    lane16 = lax.iota(jnp.int32, 16)
    ones = jnp.ones((16,), jnp.int32)
    zeros = jnp.zeros((16,), jnp.int32)

    @pl.when(wid < 5)
    def _zero_fill():
        def zstep(i, _):
            zv[pl.ds(i * 16, 16)] = jnp.zeros((16,), jnp.float32)
            return 0

        lax.fori_loop(0, CAP // 16, zstep, 0)

    for k, sh in enumerate((s_sh, x1_sh, y1_sh, x2_sh, y2_sh)):
        @pl.when(wid == k)
        def _zcp(sh=sh):
            pltpu.sync_copy(zv, sh.at[pl.ds(0, CAP)])

    BIG = jnp.int32(1 << 20)
    bigv = jnp.full((16,), BIG, jnp.int32)

    def comp_step(i, cnt):
        sl = pl.ds(i * 16, 16)
        s16 = sv[sl]
        m = (s16 >= t) & (s16 > thr)
        mi = jnp.where(m, ones, zeros)
        rankv[sl] = jnp.where(m, cnt, bigv)
        return cnt + mi

    cnt = lax.fori_loop(0, VPT, comp_step, zeros)

    n_local = jnp.int32(0)
    pvec = zeros
    for j in range(16):
        ej = cnt[j]
        n_local = n_local + ej
        pvec = pvec + jnp.where(lane16 > j, jnp.broadcast_to(ej, (16,)), zeros)

    cloc[...] = jnp.broadcast_to(n_local, (16,))
    for k in range(NSUB):
        @pl.when(wid == k)
        def _wcnt(k=k):
            pltpu.sync_copy(cloc, cnt_sh.at[pl.ds(k * 16, 16)])
    plsc.subcore_barrier()
    pltpu.sync_copy(cnt_sh, cnt_all)
    widv = jnp.broadcast_to(wid, (16,))
    gv = zeros
    for j in range(NSUB):
        cj = cnt_all[pl.ds(j * 16, 16)]
        gv = gv + cj * jnp.where(widv > j, ones, zeros)
    gp = gv + pvec

    for i in range(VPT):
        sl = pl.ds(i * 16, 16)
        r16 = rankv[sl]
        dump = jnp.full((16,), CAP + i * 16, jnp.int32) + lane16
        dst = jnp.where(r16 < bigv, gp + r16, dump)
        idxv[i // 8, pl.ds((i % 8) * 16, 16)] = dst

    copies = []
    for src, dst in ((sv, s_sh), (x1v, x1_sh), (y1v, y1_sh),
                     (x2v, x2_sh), (y2v, y2_sh)):
        for c in range(SLICE // 128):
            copies.append(
                pltpu.async_copy(src.at[pl.ds(c * 128, 128)],
                                 dst.at[idxv.at[c]], sem))
    for cp in copies:
        cp.wait()

    plsc.subcore_barrier()

    for k, (sh, out) in enumerate(((s_sh, s_out), (x1_sh, x1_out),
                                   (y1_sh, y1_out), (x2_sh, x2_out),
                                   (y2_sh, y2_out))):
        @pl.when(wid == k)
        def _flush(sh=sh, out=out):
            pltpu.sync_copy(sh.at[pl.ds(0, CAP)], out.at[pl.ds(0, CAP)])


def _nms_kernel(x1_ref, y1_ref, x2_ref, y2_ref, s_ref, data_ref,
                x1s_ref, y1s_ref, x2s_ref, y2s_ref, out_ref):
    x1 = x1_ref[...]
    y1 = y1_ref[...]
    x2 = x2_ref[...]
    y2 = y2_ref[...]
    s = s_ref[...]
    sm0 = jnp.where(s > SCORE_THRESH, s, -1.0)

    areas = (x2 - x1) * (y2 - y1)
    ii = lax.broadcasted_iota(jnp.int32, (CROWS, LANES), 0)
    jj = lax.broadcasted_iota(jnp.int32, (CROWS, LANES), 1)
    keyf = (ii * LANES + jj).astype(jnp.float32)
    BIGF = jnp.float32(CAP - 1)

    def nms_step(k, sm):
        m_b = jnp.max(jnp.max(sm, axis=1, keepdims=True), axis=0,
                      keepdims=True)
        eqv = (sm == m_b) & (m_b > 0.0)
        keym = jnp.where(eqv, keyf, BIGF)
        kmin = jnp.min(jnp.min(keym, axis=1, keepdims=True), axis=0,
                       keepdims=True)
        p = kmin[0, 0].astype(jnp.int32)
        bx1 = x1s_ref[p]
        by1 = y1s_ref[p]
        bx2 = x2s_ref[p]
        by2 = y2s_ref[p]

        xx1 = jnp.maximum(bx1, x1)
        yy1 = jnp.maximum(by1, y1)
        xx2 = jnp.minimum(bx2, x2)
        yy2 = jnp.minimum(by2, y2)
        inter = jnp.maximum(xx2 - xx1, 0.0) * jnp.maximum(yy2 - yy1, 0.0)
        barea = (bx2 - bx1) * (by2 - by1)
        denom = barea + areas - inter + 1e-9
        sm = jnp.where(2.0 * inter < denom, sm, -1.0)

        row8 = data_ref[pl.ds(p, 1), :]
        out_ref[pl.ds(k, 1), :] = row8
        return sm

    lax.fori_loop(0, MAX_DET, nms_step, sm0)


def _tile_transpose(a):
    return a.reshape(NSUB, 16, VPT).transpose(0, 2, 1).reshape(NPAD)


def kernel(boxes, scores):
    pad = NPAD - N
    s_flat = jnp.concatenate([scores, jnp.full((pad,), -1.0, jnp.float32)])
    b = jnp.concatenate([boxes, jnp.zeros((pad, 4), jnp.float32)], axis=0)

    tq = pl.pallas_call(
        _thresh_kernel,
        out_shape=jax.ShapeDtypeStruct((8, LANES), jnp.int32),
    )(s_flat.reshape(ROWS, LANES))
    t16 = jnp.broadcast_to(lax.bitcast_convert_type(tq[0, 0], jnp.float32), (16,))

    mesh = plsc.VectorSubcoreMesh(
        core_axis_name="c", subcore_axis_name="s", num_cores=1)
    f32 = jnp.float32
    sc_outs = pl.kernel(
        _sc_compact,
        out_type=[jax.ShapeDtypeStruct((OUTN,), f32)] * 5,
        mesh=mesh,
        scratch_types=[pltpu.VMEM((SLICE,), f32)] * 5 + [
            pltpu.VMEM((16,), jnp.float32),
            pltpu.VMEM((SLICE,), jnp.int32),
            pltpu.VMEM((SLICE // 128, 128), jnp.int32),
            pltpu.VMEM((16,), jnp.int32),
            pltpu.VMEM((16,), jnp.int32),
            pltpu.VMEM((NSUB * 16,), jnp.int32),
            pltpu.VMEM((CAP,), f32),
        ] + [pltpu.VMEM_SHARED((OUTN,), f32)] * 5 + [
            pltpu.VMEM_SHARED((NSUB * 16,), jnp.int32),
            pltpu.SemaphoreType.DMA,
        ],
    )(_tile_transpose(s_flat), _tile_transpose(b[:, 0]),
      _tile_transpose(b[:, 1]), _tile_transpose(b[:, 2]),
      _tile_transpose(b[:, 3]), t16)
    s_c, x1_c, y1_c, x2_c, y2_c = [a[:CAP].reshape(CROWS, LANES)
                                   for a in sc_outs]
    cols = [sc_outs[1][:CAP], sc_outs[2][:CAP], sc_outs[3][:CAP],
            sc_outs[4][:CAP], sc_outs[0][:CAP]]
    data8 = jnp.stack(cols + [jnp.zeros((CAP,), f32)] * 3, axis=1)

    vmem = pl.BlockSpec(memory_space=pltpu.MemorySpace.VMEM)
    smem = pl.BlockSpec(memory_space=pltpu.MemorySpace.SMEM)
    out = pl.pallas_call(
        _nms_kernel,
        out_shape=jax.ShapeDtypeStruct((304, 8), jnp.float32),
        in_specs=[vmem] * 6 + [smem] * 4,
    )(x1_c, y1_c, x2_c, y2_c, s_c, data8,
      sc_outs[1][:CAP], sc_outs[2][:CAP], sc_outs[3][:CAP], sc_outs[4][:CAP])
    return out[:MAX_DET, :5]


# speedup vs baseline: 1.5594x; 1.0198x over previous
import jax
import jax.numpy as jnp
from jax import lax
from jax.experimental import pallas as pl
from jax.experimental.pallas import tpu as pltpu
from jax.experimental.pallas import tpu_sc as plsc

N = 20000
NPAD = 160 * 128
ROWS = 160
LANES = 128
PRE_NMS_TOPK = 1000
MAX_DET = 300
IOU_THRESH = 0.5
SCORE_THRESH = 0.05

NSUB = 16
SLICE = NPAD // NSUB
VPT = SLICE // 16
CAP = 2048
OUTN = 2 * CAP
CROWS = CAP // LANES


def _thresh_kernel(s_ref, t_ref):
    s = s_ref[...]
    sb = lax.bitcast_convert_type(s, jnp.int32)

    def bs_step(_, carry):
        lo, hi = carry
        mid = lo + ((hi - lo) >> 1)
        cnt = jnp.sum((sb >= mid).astype(jnp.int32))
        ge = cnt >= PRE_NMS_TOPK
        lo = jnp.where(ge, mid, lo)
        hi = jnp.where(ge, hi, mid)
        return lo, hi

    lo, _ = lax.fori_loop(0, 31, bs_step, (jnp.int32(0), jnp.int32(0x7F800000)))
    t_ref[...] = jnp.full((8, LANES), lo, jnp.int32)


def _sc_compact(s_hbm, x1_hbm, y1_hbm, x2_hbm, y2_hbm, t_hbm,
                s_out, x1_out, y1_out, x2_out, y2_out,
                sv, x1v, y1v, x2v, y2v,
                tv, rankv, idxv, cntbuf, cloc, cnt_all, zv,
                s_sh, x1_sh, y1_sh, x2_sh, y2_sh, cnt_sh, sem):
    wid = lax.axis_index("s")
    base = wid * SLICE
    pltpu.sync_copy(s_hbm.at[pl.ds(base, SLICE)], sv)
    pltpu.sync_copy(x1_hbm.at[pl.ds(base, SLICE)], x1v)
    pltpu.sync_copy(y1_hbm.at[pl.ds(base, SLICE)], y1v)
    pltpu.sync_copy(x2_hbm.at[pl.ds(base, SLICE)], x2v)
    pltpu.sync_copy(y2_hbm.at[pl.ds(base, SLICE)], y2v)
    pltpu.sync_copy(t_hbm, tv)
    t = tv[...]
    thr = jnp.full((16,), SCORE_THRESH, jnp.float32)
    lane16 = lax.iota(jnp.int32, 16)
    ones = jnp.ones((16,), jnp.int32)
    zeros = jnp.zeros((16,), jnp.int32)

    @pl.when(wid < 5)
    def _zero_fill():
        def zstep(i, _):
            zv[pl.ds(i * 16, 16)] = jnp.zeros((16,), jnp.float32)
            return 0

        lax.fori_loop(0, CAP // 16, zstep, 0)

    for k, sh in enumerate((s_sh, x1_sh, y1_sh, x2_sh, y2_sh)):
        @pl.when(wid == k)
        def _zcp(sh=sh):
            pltpu.sync_copy(zv, sh.at[pl.ds(0, CAP)])

    BIG = jnp.int32(1 << 20)
    bigv = jnp.full((16,), BIG, jnp.int32)

    def comp_step(i, cnt):
        sl = pl.ds(i * 16, 16)
        s16 = sv[sl]
        m = (s16 >= t) & (s16 > thr)
        mi = jnp.where(m, ones, zeros)
        rankv[sl] = jnp.where(m, cnt, bigv)
        return cnt + mi

    cnt = lax.fori_loop(0, VPT, comp_step, zeros)

    n_local = jnp.int32(0)
    pvec = zeros
    for j in range(16):
        ej = cnt[j]
        n_local = n_local + ej
        pvec = pvec + jnp.where(lane16 > j, jnp.broadcast_to(ej, (16,)), zeros)

    cloc[...] = jnp.broadcast_to(n_local, (16,))
    for k in range(NSUB):
        @pl.when(wid == k)
        def _wcnt(k=k):
            pltpu.sync_copy(cloc, cnt_sh.at[pl.ds(k * 16, 16)])
    plsc.subcore_barrier()
    pltpu.sync_copy(cnt_sh, cnt_all)
    widv = jnp.broadcast_to(wid, (16,))
    gv = zeros
    for j in range(NSUB):
        cj = cnt_all[pl.ds(j * 16, 16)]
        gv = gv + cj * jnp.where(widv > j, ones, zeros)
    gp = gv + pvec

    for i in range(VPT):
        sl = pl.ds(i * 16, 16)
        r16 = rankv[sl]
        dump = jnp.full((16,), CAP + i * 16, jnp.int32) + lane16
        dst = jnp.where(r16 < bigv, gp + r16, dump)
        idxv[i // 8, pl.ds((i % 8) * 16, 16)] = dst

    copies = []
    for src, dst in ((sv, s_sh), (x1v, x1_sh), (y1v, y1_sh),
                     (x2v, x2_sh), (y2v, y2_sh)):
        for c in range(SLICE // 128):
            copies.append(
                pltpu.async_copy(src.at[pl.ds(c * 128, 128)],
                                 dst.at[idxv.at[c]], sem))
    for cp in copies:
        cp.wait()

    plsc.subcore_barrier()

    for k, (sh, out) in enumerate(((s_sh, s_out), (x1_sh, x1_out),
                                   (y1_sh, y1_out), (x2_sh, x2_out),
                                   (y2_sh, y2_out))):
        @pl.when(wid == k)
        def _flush(sh=sh, out=out):
            pltpu.sync_copy(sh.at[pl.ds(0, CAP)], out.at[pl.ds(0, CAP)])


def _nms_kernel(x1_ref, y1_ref, x2_ref, y2_ref, s_ref,
                x1s_ref, y1s_ref, x2s_ref, y2s_ref, ss_ref, out_ref):
    x1 = x1_ref[...]
    y1 = y1_ref[...]
    x2 = x2_ref[...]
    y2 = y2_ref[...]
    s = s_ref[...]
    sm0 = jnp.where(s > SCORE_THRESH, s, -1.0)

    areas = (x2 - x1) * (y2 - y1)
    ii = lax.broadcasted_iota(jnp.int32, (CROWS, LANES), 0)
    jj = lax.broadcasted_iota(jnp.int32, (CROWS, LANES), 1)
    keyf = (ii * LANES + jj).astype(jnp.float32)
    BIGF = jnp.float32(CAP - 1)
    lane8 = lax.broadcasted_iota(jnp.int32, (1, 8), 1)

    def nms_step(k, sm):
        m_b = jnp.max(jnp.max(sm, axis=1, keepdims=True), axis=0,
                      keepdims=True)
        eqv = (sm == m_b) & (m_b > 0.0)
        keym = jnp.where(eqv, keyf, BIGF)
        kmin = jnp.min(jnp.min(keym, axis=1, keepdims=True), axis=0,
                       keepdims=True)
        p = kmin[0, 0].astype(jnp.int32)
        bx1 = x1s_ref[p]
        by1 = y1s_ref[p]
        bx2 = x2s_ref[p]
        by2 = y2s_ref[p]
        bsc = ss_ref[p]

        xx1 = jnp.maximum(bx1, x1)
        yy1 = jnp.maximum(by1, y1)
        xx2 = jnp.minimum(bx2, x2)
        yy2 = jnp.minimum(by2, y2)
        inter = jnp.maximum(xx2 - xx1, 0.0) * jnp.maximum(yy2 - yy1, 0.0)
        barea = (bx2 - bx1) * (by2 - by1)
        denom = barea + areas - inter + 1e-9
        sm = jnp.where(2.0 * inter < denom, sm, -1.0)

        row = jnp.where(lane8 == 0, bx1, 0.0)
        row = jnp.where(lane8 == 1, by1, row)
        row = jnp.where(lane8 == 2, bx2, row)
        row = jnp.where(lane8 == 3, by2, row)
        row = jnp.where(lane8 == 4, bsc, row)
        out_ref[pl.ds(k, 1), :] = row
        return sm

    lax.fori_loop(0, MAX_DET, nms_step, sm0)


def _tile_transpose(a):
    return a.reshape(NSUB, 16, VPT).transpose(0, 2, 1).reshape(NPAD)


def kernel(boxes, scores):
    pad = NPAD - N
    s_flat = jnp.concatenate([scores, jnp.full((pad,), -1.0, jnp.float32)])
    b = jnp.concatenate([boxes, jnp.zeros((pad, 4), jnp.float32)], axis=0)

    tq = pl.pallas_call(
        _thresh_kernel,
        out_shape=jax.ShapeDtypeStruct((8, LANES), jnp.int32),
    )(s_flat.reshape(ROWS, LANES))
    t16 = jnp.broadcast_to(lax.bitcast_convert_type(tq[0, 0], jnp.float32), (16,))

    mesh = plsc.VectorSubcoreMesh(
        core_axis_name="c", subcore_axis_name="s", num_cores=1)
    f32 = jnp.float32
    sc_outs = pl.kernel(
        _sc_compact,
        out_type=[jax.ShapeDtypeStruct((OUTN,), f32)] * 5,
        mesh=mesh,
        scratch_types=[pltpu.VMEM((SLICE,), f32)] * 5 + [
            pltpu.VMEM((16,), jnp.float32),
            pltpu.VMEM((SLICE,), jnp.int32),
            pltpu.VMEM((SLICE // 128, 128), jnp.int32),
            pltpu.VMEM((16,), jnp.int32),
            pltpu.VMEM((16,), jnp.int32),
            pltpu.VMEM((NSUB * 16,), jnp.int32),
            pltpu.VMEM((CAP,), f32),
        ] + [pltpu.VMEM_SHARED((OUTN,), f32)] * 5 + [
            pltpu.VMEM_SHARED((NSUB * 16,), jnp.int32),
            pltpu.SemaphoreType.DMA,
        ],
    )(_tile_transpose(s_flat), _tile_transpose(b[:, 0]),
      _tile_transpose(b[:, 1]), _tile_transpose(b[:, 2]),
      _tile_transpose(b[:, 3]), t16)
    s_c, x1_c, y1_c, x2_c, y2_c = [a[:CAP].reshape(CROWS, LANES)
                                   for a in sc_outs]

    vmem = pl.BlockSpec(memory_space=pltpu.MemorySpace.VMEM)
    smem = pl.BlockSpec(memory_space=pltpu.MemorySpace.SMEM)
    out = pl.pallas_call(
        _nms_kernel,
        out_shape=jax.ShapeDtypeStruct((304, 8), jnp.float32),
        in_specs=[vmem] * 5 + [smem] * 5,
    )(x1_c, y1_c, x2_c, y2_c, s_c,
      sc_outs[1][:CAP], sc_outs[2][:CAP], sc_outs[3][:CAP], sc_outs[4][:CAP],
      sc_outs[0][:CAP])
    return out[:MAX_DET, :5]


# speedup vs baseline: 1.5837x; 1.0156x over previous
import jax
import jax.numpy as jnp
from jax import lax
from jax.experimental import pallas as pl
from jax.experimental.pallas import tpu as pltpu
from jax.experimental.pallas import tpu_sc as plsc

N = 20000
NPAD = 160 * 128
ROWS = 160
LANES = 128
PRE_NMS_TOPK = 1000
MAX_DET = 300
IOU_THRESH = 0.5
SCORE_THRESH = 0.05

NSUB = 16
SLICE = NPAD // NSUB
VPT = SLICE // 16
CAP = 1024
OUTN = CAP + SLICE
CROWS = CAP // LANES


def _thresh_kernel(s_ref, t_ref):
    s = s_ref[...]
    sb = lax.bitcast_convert_type(s, jnp.int32)

    def bs_step(_, carry):
        lo, hi = carry
        mid = lo + ((hi - lo) >> 1)
        cnt = jnp.sum((sb >= mid).astype(jnp.int32))
        ge = cnt >= PRE_NMS_TOPK
        lo = jnp.where(ge, mid, lo)
        hi = jnp.where(ge, hi, mid)
        return lo, hi

    lo, _ = lax.fori_loop(0, 31, bs_step, (jnp.int32(0), jnp.int32(0x7F800000)))
    t_ref[...] = jnp.full((8, LANES), lo, jnp.int32)


def _sc_compact(s_hbm, x1_hbm, y1_hbm, x2_hbm, y2_hbm, t_hbm,
                s_out, x1_out, y1_out, x2_out, y2_out,
                sv, x1v, y1v, x2v, y2v,
                tv, rankv, idxv, cntbuf, cloc, cnt_all, zv,
                s_sh, x1_sh, y1_sh, x2_sh, y2_sh, cnt_sh, sem):
    wid = lax.axis_index("s")
    base = wid * SLICE
    pltpu.sync_copy(s_hbm.at[pl.ds(base, SLICE)], sv)
    pltpu.sync_copy(x1_hbm.at[pl.ds(base, SLICE)], x1v)
    pltpu.sync_copy(y1_hbm.at[pl.ds(base, SLICE)], y1v)
    pltpu.sync_copy(x2_hbm.at[pl.ds(base, SLICE)], x2v)
    pltpu.sync_copy(y2_hbm.at[pl.ds(base, SLICE)], y2v)
    pltpu.sync_copy(t_hbm, tv)
    t = tv[...]
    thr = jnp.full((16,), SCORE_THRESH, jnp.float32)
    lane16 = lax.iota(jnp.int32, 16)
    ones = jnp.ones((16,), jnp.int32)
    zeros = jnp.zeros((16,), jnp.int32)

    @pl.when(wid < 5)
    def _zero_fill():
        def zstep(i, _):
            zv[pl.ds(i * 16, 16)] = jnp.zeros((16,), jnp.float32)
            return 0

        lax.fori_loop(0, CAP // 16, zstep, 0)

    for k, sh in enumerate((s_sh, x1_sh, y1_sh, x2_sh, y2_sh)):
        @pl.when(wid == k)
        def _zcp(sh=sh):
            pltpu.sync_copy(zv, sh.at[pl.ds(0, CAP)])

    BIG = jnp.int32(1 << 20)
    bigv = jnp.full((16,), BIG, jnp.int32)

    def comp_step(i, cnt):
        sl = pl.ds(i * 16, 16)
        s16 = sv[sl]
        m = (s16 >= t) & (s16 > thr)
        mi = jnp.where(m, ones, zeros)
        rankv[sl] = jnp.where(m, cnt, bigv)
        return cnt + mi

    cnt = lax.fori_loop(0, VPT, comp_step, zeros)

    n_local = jnp.int32(0)
    pvec = zeros
    for j in range(16):
        ej = cnt[j]
        n_local = n_local + ej
        pvec = pvec + jnp.where(lane16 > j, jnp.broadcast_to(ej, (16,)), zeros)

    cloc[...] = jnp.broadcast_to(n_local, (16,))
    for k in range(NSUB):
        @pl.when(wid == k)
        def _wcnt(k=k):
            pltpu.sync_copy(cloc, cnt_sh.at[pl.ds(k * 16, 16)])
    plsc.subcore_barrier()
    pltpu.sync_copy(cnt_sh, cnt_all)
    widv = jnp.broadcast_to(wid, (16,))
    gv = zeros
    for j in range(NSUB):
        cj = cnt_all[pl.ds(j * 16, 16)]
        gv = gv + cj * jnp.where(widv > j, ones, zeros)
    gp = gv + pvec

    for i in range(VPT):
        sl = pl.ds(i * 16, 16)
        r16 = rankv[sl]
        dump = jnp.full((16,), CAP + i * 16, jnp.int32) + lane16
        dst = jnp.where(r16 < bigv, gp + r16, dump)
        idxv[i // 8, pl.ds((i % 8) * 16, 16)] = dst

    copies = []
    for src, dst in ((sv, s_sh), (x1v, x1_sh), (y1v, y1_sh),
                     (x2v, x2_sh), (y2v, y2_sh)):
        for c in range(SLICE // 128):
            copies.append(
                pltpu.async_copy(src.at[pl.ds(c * 128, 128)],
                                 dst.at[idxv.at[c]], sem))
    for cp in copies:
        cp.wait()

    plsc.subcore_barrier()

    for k, (sh, out) in enumerate(((s_sh, s_out), (x1_sh, x1_out),
                                   (y1_sh, y1_out), (x2_sh, x2_out),
                                   (y2_sh, y2_out))):
        @pl.when(wid == k)
        def _flush(sh=sh, out=out):
            pltpu.sync_copy(sh.at[pl.ds(0, CAP)], out.at[pl.ds(0, CAP)])


def _nms_kernel(x1_ref, y1_ref, x2_ref, y2_ref, s_ref,
                x1s_ref, y1s_ref, x2s_ref, y2s_ref, ss_ref, out_ref):
    x1 = x1_ref[...]
    y1 = y1_ref[...]
    x2 = x2_ref[...]
    y2 = y2_ref[...]
    s = s_ref[...]
    sm0 = jnp.where(s > SCORE_THRESH, s, -1.0)

    areas = (x2 - x1) * (y2 - y1)
    ii = lax.broadcasted_iota(jnp.int32, (CROWS, LANES), 0)
    jj = lax.broadcasted_iota(jnp.int32, (CROWS, LANES), 1)
    keyf = (ii * LANES + jj).astype(jnp.float32)
    BIGF = jnp.float32(CAP - 1)
    lane8 = lax.broadcasted_iota(jnp.int32, (1, 8), 1)

    def nms_step(k, sm):
        m_b = jnp.max(jnp.max(sm, axis=1, keepdims=True), axis=0,
                      keepdims=True)
        eqv = (sm == m_b) & (m_b > 0.0)
        keym = jnp.where(eqv, keyf, BIGF)
        kmin = jnp.min(jnp.min(keym, axis=1, keepdims=True), axis=0,
                       keepdims=True)
        p = kmin[0, 0].astype(jnp.int32)
        bx1 = x1s_ref[p]
        by1 = y1s_ref[p]
        bx2 = x2s_ref[p]
        by2 = y2s_ref[p]
        bsc = ss_ref[p]

        xx1 = jnp.maximum(bx1, x1)
        yy1 = jnp.maximum(by1, y1)
        xx2 = jnp.minimum(bx2, x2)
        yy2 = jnp.minimum(by2, y2)
        inter = jnp.maximum(xx2 - xx1, 0.0) * jnp.maximum(yy2 - yy1, 0.0)
        barea = (bx2 - bx1) * (by2 - by1)
        denom = barea + areas - inter + 1e-9
        sm = jnp.where(2.0 * inter < denom, sm, -1.0)

        row = jnp.where(lane8 == 0, bx1, 0.0)
        row = jnp.where(lane8 == 1, by1, row)
        row = jnp.where(lane8 == 2, bx2, row)
        row = jnp.where(lane8 == 3, by2, row)
        row = jnp.where(lane8 == 4, bsc, row)
        out_ref[pl.ds(k, 1), :] = row
        return sm

    lax.fori_loop(0, MAX_DET, nms_step, sm0)


def _tile_transpose(a):
    return a.reshape(NSUB, 16, VPT).transpose(0, 2, 1).reshape(NPAD)


def kernel(boxes, scores):
    pad = NPAD - N
    s_flat = jnp.concatenate([scores, jnp.full((pad,), -1.0, jnp.float32)])
    b = jnp.concatenate([boxes, jnp.zeros((pad, 4), jnp.float32)], axis=0)

    tq = pl.pallas_call(
        _thresh_kernel,
        out_shape=jax.ShapeDtypeStruct((8, LANES), jnp.int32),
    )(s_flat.reshape(ROWS, LANES))
    t16 = jnp.broadcast_to(lax.bitcast_convert_type(tq[0, 0], jnp.float32), (16,))

    mesh = plsc.VectorSubcoreMesh(
        core_axis_name="c", subcore_axis_name="s", num_cores=1)
    f32 = jnp.float32
    sc_outs = pl.kernel(
        _sc_compact,
        out_type=[jax.ShapeDtypeStruct((OUTN,), f32)] * 5,
        mesh=mesh,
        scratch_types=[pltpu.VMEM((SLICE,), f32)] * 5 + [
            pltpu.VMEM((16,), jnp.float32),
            pltpu.VMEM((SLICE,), jnp.int32),
            pltpu.VMEM((SLICE // 128, 128), jnp.int32),
            pltpu.VMEM((16,), jnp.int32),
            pltpu.VMEM((16,), jnp.int32),
            pltpu.VMEM((NSUB * 16,), jnp.int32),
            pltpu.VMEM((CAP,), f32),
        ] + [pltpu.VMEM_SHARED((OUTN,), f32)] * 5 + [
            pltpu.VMEM_SHARED((NSUB * 16,), jnp.int32),
            pltpu.SemaphoreType.DMA,
        ],
    )(_tile_transpose(s_flat), _tile_transpose(b[:, 0]),
      _tile_transpose(b[:, 1]), _tile_transpose(b[:, 2]),
      _tile_transpose(b[:, 3]), t16)
    s_c, x1_c, y1_c, x2_c, y2_c = [a[:CAP].reshape(CROWS, LANES)
                                   for a in sc_outs]

    vmem = pl.BlockSpec(memory_space=pltpu.MemorySpace.VMEM)
    smem = pl.BlockSpec(memory_space=pltpu.MemorySpace.SMEM)
    out = pl.pallas_call(
        _nms_kernel,
        out_shape=jax.ShapeDtypeStruct((304, 8), jnp.float32),
        in_specs=[vmem] * 5 + [smem] * 5,
    )(x1_c, y1_c, x2_c, y2_c, s_c,
      sc_outs[1][:CAP], sc_outs[2][:CAP], sc_outs[3][:CAP], sc_outs[4][:CAP],
      sc_outs[0][:CAP])
    return out[:MAX_DET, :5]
